# baseline (speedup 1.0000x reference)
import functools
import jax
import jax.numpy as jnp
from jax import lax
from jax.experimental import pallas as pl
from jax.experimental.pallas import tpu as pltpu
from jax.experimental.pallas import tpu_sc as plsc

EDIM = 64
LNBR = 50
NC = 2
NS = 16
NW = NC * NS
NLANE = 16
NBUF = 4
OCH = 32


def _sc_feats(user_embs, item_embs, uid, nbr, uiu, uui, wpack, gx):
    B = uid.shape[0]
    assert B % (NW * NBUF) == 0
    n_per = B // NW
    f32 = jnp.float32
    i32 = jnp.int32
    mesh = plsc.VectorSubcoreMesh(
        core_axis_name="c", subcore_axis_name="s",
        num_cores=NC, num_subcores=NS)

    @functools.partial(
        pl.kernel,
        out_type=[jax.ShapeDtypeStruct((B, EDIM), f32) for _ in range(3)],
        mesh=mesh,
        compiler_params=pltpu.CompilerParams(
            needs_layout_passes=False, use_tc_tiling_on_sc=False),
        scratch_types=[
            pltpu.VMEM((n_per, LNBR), i32),
            pltpu.VMEM((n_per, 2, LNBR), i32),
            pltpu.VMEM((n_per, 2, LNBR), i32),
            pltpu.VMEM((n_per,), i32),
            pltpu.VMEM((n_per, EDIM), f32),
            pltpu.VMEM((8, 4, NLANE), f32),
            pltpu.VMEM((3, EDIM, NLANE), f32),
            pltpu.VMEM((4, NLANE), f32),
            pltpu.VMEM((NBUF, LNBR, EDIM), f32),
            pltpu.VMEM((NBUF, LNBR, EDIM), f32),
            pltpu.VMEM((NBUF, LNBR, EDIM), f32),
            pltpu.VMEM((NBUF, LNBR, EDIM), f32),
            pltpu.VMEM((NBUF, LNBR, EDIM), f32),
            pltpu.VMEM((3, OCH, EDIM), f32),
            pltpu.SemaphoreType.DMA,
            pltpu.SemaphoreType.DMA((NBUF,)),
            pltpu.SemaphoreType.DMA,
        ],
    )
    def k(user_hbm, item_hbm, uid_hbm, nbr_hbm, uiu_hbm, uui_hbm, wpack_hbm,
          gx_hbm, uu_out, uiu_out, uui_out,
          idx_nbr, idx_uiu, idx_uui, uid_v, selfr, wv, gxm, attnv,
          r_nbr, r_ui_i, r_ui_u, r_uu_u, r_uu_i,
          ostage, sem, dsem, osem):
        wid = lax.axis_index("s") * NC + lax.axis_index("c")
        base = wid * n_per
        pltpu.sync_copy(nbr_hbm.at[pl.ds(base, n_per)], idx_nbr)
        pltpu.sync_copy(uiu_hbm.at[pl.ds(base, n_per)], idx_uiu)
        pltpu.sync_copy(uui_hbm.at[pl.ds(base, n_per)], idx_uui)
        pltpu.sync_copy(uid_hbm.at[pl.ds(base, n_per)], uid_v)
        pltpu.sync_copy(wpack_hbm, wv)
        pltpu.sync_copy(gx_hbm, gxm)
        pltpu.async_copy(user_hbm.at[uid_v], selfr, sem).wait()

        iota = lax.iota(i32, NLANE)
        zero16 = jnp.zeros((NLANE,), f32)

        def streams(i, b):
            return (
                (user_hbm.at[idx_nbr.at[i]], r_nbr.at[b]),
                (item_hbm.at[idx_uiu.at[i, 0]], r_ui_i.at[b]),
                (user_hbm.at[idx_uiu.at[i, 1]], r_ui_u.at[b]),
                (user_hbm.at[idx_uui.at[i, 0]], r_uu_u.at[b]),
                (item_hbm.at[idx_uui.at[i, 1]], r_uu_i.at[b]),
            )

        def issue(i, b):
            for src, dst in streams(i, b):
                pltpu.async_copy(src, dst, dsem.at[b])

        def drain(i, b):
            for src, dst in streams(i, b):
                pltpu.make_async_copy(src, dst, dsem.at[b]).wait()

        outs = (uu_out, uiu_out, uui_out)

        def out_streams(i):
            return [(ostage.at[p], outs[p].at[pl.ds(base + i - (OCH - 1), OCH)])
                    for p in range(3)]

        def compute(i, b):
            io = lax.rem(i, OCH)
            @pl.when(jnp.logical_and(io == 0, i > 0))
            def _():
                for src, dst in out_streams(i - 1):
                    pltpu.make_async_copy(src, dst, osem).wait()

            sv = [selfr[i, pl.ds(16 * j, 16)] for j in range(4)]
            ifull = jnp.full((NLANE,), i, i32)
            bfull = jnp.full((NLANE,), b, i32)

            def presum(lp, carry):
                for dstb, srcb in ((r_ui_i, r_ui_u), (r_uu_u, r_uu_i)):
                    for j in range(4):
                        s = pl.ds(16 * j, 16)
                        dstb[b, lp, s] = dstb[b, lp, s] + srcb[b, lp, s]
                return carry

            lax.fori_loop(0, LNBR, presum, 0)

            for p, (bufs, midx, coef) in enumerate((
                    ((r_nbr,), None, 0.5),
                    ((r_ui_i,), idx_uiu, 1.0 / 3.0),
                    ((r_uu_u,), idx_uui, 1.0 / 3.0),
            )):
                grow = 2 * p
                urow = 2 * p + 1
                cself = jnp.float32(0.0)
                for j in range(4):
                    cself = cself + jnp.sum(sv[j] * wv[urow, j])

                rids = [jnp.minimum(iota + 16 * j, LNBR - 1) for j in range(4)]

                def s1body(c16, d, bufs=bufs, p=p, rids=rids, bfull=bfull):
                    d = list(d)
                    for cc in range(16):
                        c = 16 * c16 + cc
                        colv = jnp.bitwise_and(iota + c, EDIM - 1)
                        wrot = gxm[p, c]
                        for j in range(4):
                            g = plsc.load_gather(bufs[0],
                                                 [bfull, rids[j], colv])
                            if len(bufs) == 2:
                                g = g + plsc.load_gather(
                                    bufs[1], [bfull, rids[j], colv])
                            d[j] = d[j] + g * wrot
                    return tuple(d)

                d = list(lax.fori_loop(0, 4, s1body,
                                       (zero16, zero16, zero16, zero16)))
                for j in range(4):
                    dj = d[j] + cself
                    dj = jnp.where(dj >= 0.0, dj, 0.01 * dj)
                    if midx is None:
                        miv = plsc.load_gather(idx_nbr, [ifull, rids[j]])
                    else:
                        miv = plsc.load_gather(
                            midx, [ifull, jnp.zeros((NLANE,), i32), rids[j]])
                    dj = jnp.where(miv == 0, dj - 1e8, dj)
                    dj = jnp.where(iota + 16 * j >= LNBR, -1e30, dj)
                    d[j] = dj

                mall = jnp.max(jnp.maximum(jnp.maximum(d[0], d[1]),
                                           jnp.maximum(d[2], d[3])))
                es = [jnp.exp(dj - mall) for dj in d]
                ssum = jnp.sum(es[0] + es[1] + es[2] + es[3])
                for j in range(4):
                    attnv[j] = es[j]

                def s2body(q, o, bufs=bufs):
                    av = attnv[q]
                    o = list(o)
                    for cc in range(16):
                        l = 16 * q + cc
                        a_sc = av[cc]
                        for j in range(4):
                            t = bufs[0][b, l, pl.ds(16 * j, 16)]
                            if len(bufs) == 2:
                                t = t + bufs[1][b, l, pl.ds(16 * j, 16)]
                            o[j] = o[j] + a_sc * t
                    return tuple(o)

                o = list(lax.fori_loop(0, 3, s2body,
                                       (zero16, zero16, zero16, zero16)))
                av3 = attnv[3]
                for cc in range(LNBR - 48):
                    l3 = 48 + cc
                    a_sc = av3[cc]
                    for j in range(4):
                        t = bufs[0][b, l3, pl.ds(16 * j, 16)]
                        if len(bufs) == 2:
                            t = t + bufs[1][b, l3, pl.ds(16 * j, 16)]
                        o[j] = o[j] + a_sc * t
                rinv = (jnp.full((NLANE,), coef, f32)
                        / jnp.full((NLANE,), ssum, f32))
                for j in range(4):
                    ostage[p, io, pl.ds(16 * j, 16)] = jnp.maximum(
                        coef * sv[j] + rinv * o[j], 0.0)

            @pl.when(io == OCH - 1)
            def _():
                for src, dst in out_streams(i):
                    pltpu.async_copy(src, dst, osem)

        for b in range(NBUF):
            issue(b, b)

        def body(i, carry):
            b = lax.rem(i, NBUF)
            drain(i, b)
            compute(i, b)
            inext = i + NBUF

            @pl.when(inext < n_per)
            def _():
                issue(inext, b)
            return carry

        lax.fori_loop(0, n_per, body, 0)

        for src, dst in out_streams(n_per - 1):
            pltpu.make_async_copy(src, dst, osem).wait()

    return k(user_embs, item_embs, uid, nbr, uiu, uui, wpack, gx)


def _tc_combine(uu, uiu, uui, W0, b0, W1):
    B = uu.shape[0]

    def body(uu_ref, uiu_ref, uui_ref, w0_ref, b0_ref, w1_ref, out_ref):
        w0 = w0_ref[...]
        b0 = b0_ref[...]
        w1 = w1_ref[...]
        feats = [uu_ref[...], uiu_ref[...], uui_ref[...]]
        ss = []
        for f in feats:
            h = jnp.tanh(lax.dot_general(
                f, w0, (((1,), (1,)), ((), ())),
                preferred_element_type=jnp.float32) + b0)
            hm = jnp.mean(h, axis=0, keepdims=True)
            ss.append(jnp.sum(hm * w1))
        m = jnp.maximum(jnp.maximum(ss[0], ss[1]), ss[2])
        e = [jnp.exp(x - m) for x in ss]
        tot = e[0] + e[1] + e[2]
        out = (e[0] * feats[0] + e[1] * feats[1] + e[2] * feats[2]) / tot
        out_ref[...] = jnp.maximum(out, 0.0)

    return pl.pallas_call(
        body,
        out_shape=jax.ShapeDtypeStruct((B, EDIM), jnp.float32),
    )(uu, uiu, uui, W0, b0, W1)


def kernel(user_embs, item_embs, uid, nbr, uiu, uui, W_uu, W_uiu, W_uui, W0, b0, W1):
    E = EDIM

    def gu(W, K):
        g = W[0, E:] / K
        return g, W[0, :E] + g

    g0, u0 = gu(W_uu, 2.0)
    g1, u1 = gu(W_uiu, 3.0)
    g2, u2 = gu(W_uui, 3.0)
    zeros = jnp.zeros((E,), jnp.float32)
    wpack = jnp.stack([g0, u0, g1, u1, g2, u2, zeros, zeros]).reshape(8, 4, 16)
    rot = (jnp.arange(E)[:, None] + jnp.arange(NLANE)[None, :]) % E
    gx = jnp.stack([g0[rot], g1[rot], g2[rot]])

    uu_f, uiu_f, uui_f = _sc_feats(user_embs, item_embs, uid, nbr, uiu, uui,
                                   wpack, gx)
    return _tc_combine(uu_f, uiu_f, uui_f, W0,
                       b0.reshape(1, E // 2), W1)


# speedup vs baseline: 1.0967x; 1.0967x over previous
import functools
import numpy as np
import jax
import jax.numpy as jnp
from jax import lax
from jax.experimental import pallas as pl
from jax.experimental.pallas import tpu as pltpu
from jax.experimental.pallas import tpu_sc as plsc

EDIM = 64
LNBR = 50
NC = 2
NS = 16
NW = NC * NS
NLANE = 16
NBUF = 2
OCH = 32


def _sc_feats(user_embs, item_embs, uid, nbr, uiu, uui, wpack, gx):
    B = uid.shape[0]
    assert B % (NW * NBUF) == 0
    n_per = B // NW
    f32 = jnp.float32
    i32 = jnp.int32
    mesh = plsc.VectorSubcoreMesh(
        core_axis_name="c", subcore_axis_name="s",
        num_cores=NC, num_subcores=NS)

    @functools.partial(
        pl.kernel,
        out_type=[jax.ShapeDtypeStruct((B, EDIM), f32) for _ in range(3)],
        mesh=mesh,
        compiler_params=pltpu.CompilerParams(
            needs_layout_passes=False, use_tc_tiling_on_sc=False),
        scratch_types=[
            pltpu.VMEM((n_per, LNBR), i32),
            pltpu.VMEM((n_per, 2, LNBR), i32),
            pltpu.VMEM((n_per, 2, LNBR), i32),
            pltpu.VMEM((n_per,), i32),
            pltpu.VMEM((n_per, EDIM), f32),
            pltpu.VMEM((8, 4, NLANE), f32),
            pltpu.VMEM((3, EDIM, NLANE), f32),
            pltpu.VMEM((4, NLANE), f32),
            pltpu.VMEM((4, NLANE), f32),
            pltpu.VMEM((NBUF, LNBR, EDIM), f32),
            pltpu.VMEM((NBUF, LNBR, EDIM), f32),
            pltpu.VMEM((NBUF, LNBR, EDIM), f32),
            pltpu.VMEM((NBUF, LNBR, EDIM), f32),
            pltpu.VMEM((NBUF, LNBR, EDIM), f32),
            pltpu.VMEM((3, OCH, EDIM), f32),
            pltpu.SemaphoreType.DMA,
            pltpu.SemaphoreType.DMA((NBUF,)),
            pltpu.SemaphoreType.DMA,
        ],
    )
    def k(user_hbm, item_hbm, uid_hbm, nbr_hbm, uiu_hbm, uui_hbm, wpack_hbm,
          gx_hbm, uu_out, uiu_out, uui_out,
          idx_nbr, idx_uiu, idx_uui, uid_v, selfr, wv, gxm, attnv, dstage,
          r_nbr, r_ui_i, r_ui_u, r_uu_u, r_uu_i,
          ostage, sem, dsem, osem):
        wid = lax.axis_index("s") * NC + lax.axis_index("c")
        base = wid * n_per
        pltpu.sync_copy(nbr_hbm.at[pl.ds(base, n_per)], idx_nbr)
        pltpu.sync_copy(uiu_hbm.at[pl.ds(base, n_per)], idx_uiu)
        pltpu.sync_copy(uui_hbm.at[pl.ds(base, n_per)], idx_uui)
        pltpu.sync_copy(uid_hbm.at[pl.ds(base, n_per)], uid_v)
        pltpu.sync_copy(wpack_hbm, wv)
        pltpu.sync_copy(gx_hbm, gxm)
        pltpu.async_copy(user_hbm.at[uid_v], selfr, sem).wait()

        iota = lax.iota(i32, NLANE)
        zero16 = jnp.zeros((NLANE,), f32)
        mask8 = iota < 8

        tree_idx = []
        for r, w in ((1, 16), (2, 8), (4, 4), (8, 2)):
            h = w // 2
            kq = iota // h
            t = iota % h
            ia = jnp.minimum(kq * w + t, NLANE - 1)
            ib = jnp.clip((kq - r) * w + t, 0, NLANE - 1)
            tree_idx.append((ia, jnp.minimum(ia + h, NLANE - 1),
                             ib, jnp.minimum(ib + h, NLANE - 1)))

        def perm(x, idx):
            return jnp.take_along_axis(x, idx, axis=0,
                                       mode="promise_in_bounds")

        def lane_sums(vecs):
            for lv in range(4):
                ia, iah, ib, ibh = tree_idx[lv]
                vecs = [jnp.where(mask8,
                                  perm(vecs[2 * k], ia) + perm(vecs[2 * k], iah),
                                  perm(vecs[2 * k + 1], ib)
                                  + perm(vecs[2 * k + 1], ibh))
                        for k in range(len(vecs) // 2)]
            return vecs[0]

        def streams(i, b):
            return (
                (user_hbm.at[idx_nbr.at[i]], r_nbr.at[b]),
                (item_hbm.at[idx_uiu.at[i, 0]], r_ui_i.at[b]),
                (user_hbm.at[idx_uiu.at[i, 1]], r_ui_u.at[b]),
                (user_hbm.at[idx_uui.at[i, 0]], r_uu_u.at[b]),
                (item_hbm.at[idx_uui.at[i, 1]], r_uu_i.at[b]),
            )

        def issue(i, b):
            for src, dst in streams(i, b):
                pltpu.async_copy(src, dst, dsem.at[b])

        def drain(i, b):
            for src, dst in streams(i, b):
                pltpu.make_async_copy(src, dst, dsem.at[b]).wait()

        outs = (uu_out, uiu_out, uui_out)

        def out_streams(i):
            return [(ostage.at[p], outs[p].at[pl.ds(base + i - (OCH - 1), OCH)])
                    for p in range(3)]

        def compute(i, b):
            io = lax.rem(i, OCH)
            @pl.when(jnp.logical_and(io == 0, i > 0))
            def _():
                for src, dst in out_streams(i - 1):
                    pltpu.make_async_copy(src, dst, osem).wait()

            sv = [selfr[i, pl.ds(16 * j, 16)] for j in range(4)]
            ifull = jnp.full((NLANE,), i, i32)
            bfull = jnp.full((NLANE,), b, i32)

            def presum(lp, carry):
                for dstb, srcb in ((r_ui_i, r_ui_u), (r_uu_u, r_uu_i)):
                    for j in range(4):
                        s = pl.ds(16 * j, 16)
                        dstb[b, lp, s] = dstb[b, lp, s] + srcb[b, lp, s]
                return carry

            lax.fori_loop(0, LNBR, presum, 0)

            for p, (bufs, midx, coef) in enumerate((
                    ((r_nbr,), None, 0.5),
                    ((r_ui_i,), idx_uiu, 1.0 / 3.0),
                    ((r_uu_u,), idx_uui, 1.0 / 3.0),
            )):
                grow = 2 * p
                urow = 2 * p + 1
                cself = jnp.float32(0.0)
                for j in range(4):
                    cself = cself + jnp.sum(sv[j] * wv[urow, j])

                rids = [jnp.minimum(iota + 16 * j, LNBR - 1) for j in range(4)]

                gqs = [gxm[p, 16 * q] for q in range(4)]

                def s1body(jj, carry, buf=bufs[0], gqs=gqs):
                    vecs = []
                    for cc in range(16):
                        l = jnp.minimum(16 * jj + cc, LNBR - 1)
                        s = gqs[0] * buf[b, l, pl.ds(0, 16)]
                        for q in range(1, 4):
                            s = s + gqs[q] * buf[b, l, pl.ds(16 * q, 16)]
                        vecs.append(s)
                    dstage[jj] = lane_sums(vecs)
                    return carry

                lax.fori_loop(0, 4, s1body, 0)
                d = [dstage[j] for j in range(4)]
                for j in range(4):
                    dj = d[j] + cself
                    dj = jnp.where(dj >= 0.0, dj, 0.01 * dj)
                    if midx is None:
                        miv = plsc.load_gather(idx_nbr, [ifull, rids[j]])
                    else:
                        miv = plsc.load_gather(
                            midx, [ifull, jnp.zeros((NLANE,), i32), rids[j]])
                    dj = jnp.where(miv == 0, dj - 1e8, dj)
                    dj = jnp.where(iota + 16 * j >= LNBR, -1e30, dj)
                    d[j] = dj

                mall = jnp.max(jnp.maximum(jnp.maximum(d[0], d[1]),
                                           jnp.maximum(d[2], d[3])))
                es = [jnp.exp(dj - mall) for dj in d]
                ssum = jnp.sum(es[0] + es[1] + es[2] + es[3])
                for j in range(4):
                    attnv[j] = es[j]

                def s2body(q, o, bufs=bufs):
                    av = attnv[q]
                    o = list(o)
                    for cc in range(16):
                        l = 16 * q + cc
                        a_sc = av[cc]
                        for j in range(4):
                            t = bufs[0][b, l, pl.ds(16 * j, 16)]
                            if len(bufs) == 2:
                                t = t + bufs[1][b, l, pl.ds(16 * j, 16)]
                            o[j] = o[j] + a_sc * t
                    return tuple(o)

                o = list(lax.fori_loop(0, 3, s2body,
                                       (zero16, zero16, zero16, zero16)))
                av3 = attnv[3]
                for cc in range(LNBR - 48):
                    l3 = 48 + cc
                    a_sc = av3[cc]
                    for j in range(4):
                        t = bufs[0][b, l3, pl.ds(16 * j, 16)]
                        if len(bufs) == 2:
                            t = t + bufs[1][b, l3, pl.ds(16 * j, 16)]
                        o[j] = o[j] + a_sc * t
                rinv = (jnp.full((NLANE,), coef, f32)
                        / jnp.full((NLANE,), ssum, f32))
                for j in range(4):
                    ostage[p, io, pl.ds(16 * j, 16)] = jnp.maximum(
                        coef * sv[j] + rinv * o[j], 0.0)

            @pl.when(io == OCH - 1)
            def _():
                for src, dst in out_streams(i):
                    pltpu.async_copy(src, dst, osem)

        for b in range(NBUF):
            issue(b, b)

        def body(i, carry):
            b = lax.rem(i, NBUF)
            drain(i, b)
            compute(i, b)
            inext = i + NBUF

            @pl.when(inext < n_per)
            def _():
                issue(inext, b)
            return carry

        lax.fori_loop(0, n_per, body, 0)

        for src, dst in out_streams(n_per - 1):
            pltpu.make_async_copy(src, dst, osem).wait()

    return k(user_embs, item_embs, uid, nbr, uiu, uui, wpack, gx)


def _tc_combine(uu, uiu, uui, W0, b0, W1):
    B = uu.shape[0]

    def body(uu_ref, uiu_ref, uui_ref, w0_ref, b0_ref, w1_ref, out_ref):
        w0 = w0_ref[...]
        b0 = b0_ref[...]
        w1 = w1_ref[...]
        feats = [uu_ref[...], uiu_ref[...], uui_ref[...]]
        ss = []
        for f in feats:
            h = jnp.tanh(lax.dot_general(
                f, w0, (((1,), (1,)), ((), ())),
                preferred_element_type=jnp.float32) + b0)
            hm = jnp.mean(h, axis=0, keepdims=True)
            ss.append(jnp.sum(hm * w1))
        m = jnp.maximum(jnp.maximum(ss[0], ss[1]), ss[2])
        e = [jnp.exp(x - m) for x in ss]
        tot = e[0] + e[1] + e[2]
        out = (e[0] * feats[0] + e[1] * feats[1] + e[2] * feats[2]) / tot
        out_ref[...] = jnp.maximum(out, 0.0)

    return pl.pallas_call(
        body,
        out_shape=jax.ShapeDtypeStruct((B, EDIM), jnp.float32),
    )(uu, uiu, uui, W0, b0, W1)


def kernel(user_embs, item_embs, uid, nbr, uiu, uui, W_uu, W_uiu, W_uui, W0, b0, W1):
    E = EDIM

    def gu(W, K):
        g = W[0, E:] / K
        return g, W[0, :E] + g

    g0, u0 = gu(W_uu, 2.0)
    g1, u1 = gu(W_uiu, 3.0)
    g2, u2 = gu(W_uui, 3.0)
    zeros = jnp.zeros((E,), jnp.float32)
    wpack = jnp.stack([g0, u0, g1, u1, g2, u2, zeros, zeros]).reshape(8, 4, 16)
    rot = (jnp.arange(E)[:, None] + jnp.arange(NLANE)[None, :]) % E
    gx = jnp.stack([g0[rot], g1[rot], g2[rot]])

    uu_f, uiu_f, uui_f = _sc_feats(user_embs, item_embs, uid, nbr, uiu, uui,
                                   wpack, gx)
    return _tc_combine(uu_f, uiu_f, uui_f, W0,
                       b0.reshape(1, E // 2), W1)


# speedup vs baseline: 1.0973x; 1.0005x over previous
import functools
import jax
import jax.numpy as jnp
from jax import lax
from jax.experimental import pallas as pl
from jax.experimental.pallas import tpu as pltpu
from jax.experimental.pallas import tpu_sc as plsc

EDIM = 64
LNBR = 50
NC = 2
NS = 16
NW = NC * NS
NLANE = 16
NBUF = 2
OCH = 32


def _sc_feats(user_embs, item_embs, uid, nbr, uiu, uui, wpack, gx):
    B = uid.shape[0]
    assert B % (NW * NBUF) == 0
    n_per = B // NW
    f32 = jnp.float32
    i32 = jnp.int32
    mesh = plsc.VectorSubcoreMesh(
        core_axis_name="c", subcore_axis_name="s",
        num_cores=NC, num_subcores=NS)

    @functools.partial(
        pl.kernel,
        out_type=[jax.ShapeDtypeStruct((B, EDIM), f32) for _ in range(3)],
        mesh=mesh,
        compiler_params=pltpu.CompilerParams(
            needs_layout_passes=False, use_tc_tiling_on_sc=False),
        scratch_types=[
            pltpu.VMEM((n_per, LNBR), i32),
            pltpu.VMEM((n_per, 2, LNBR), i32),
            pltpu.VMEM((n_per, 2, LNBR), i32),
            pltpu.VMEM((n_per,), i32),
            pltpu.VMEM((n_per, EDIM), f32),
            pltpu.VMEM((8, 4, NLANE), f32),
            pltpu.VMEM((3, EDIM, NLANE), f32),
            pltpu.VMEM((4, NLANE), f32),
            pltpu.VMEM((4, NLANE), f32),
            pltpu.VMEM((NBUF, LNBR, EDIM), f32),
            pltpu.VMEM((NBUF, LNBR, EDIM), f32),
            pltpu.VMEM((NBUF, LNBR, EDIM), f32),
            pltpu.VMEM((NBUF, LNBR, EDIM), f32),
            pltpu.VMEM((NBUF, LNBR, EDIM), f32),
            pltpu.VMEM((3, OCH, EDIM), f32),
            pltpu.SemaphoreType.DMA,
            pltpu.SemaphoreType.DMA((NBUF,)),
            pltpu.SemaphoreType.DMA,
        ],
    )
    def k(user_hbm, item_hbm, uid_hbm, nbr_hbm, uiu_hbm, uui_hbm, wpack_hbm,
          gx_hbm, uu_out, uiu_out, uui_out,
          idx_nbr, idx_uiu, idx_uui, uid_v, selfr, wv, gxm, attnv, dstage,
          r_nbr, r_ui_i, r_ui_u, r_uu_u, r_uu_i,
          ostage, sem, dsem, osem):
        wid = lax.axis_index("s") * NC + lax.axis_index("c")
        base = wid * n_per
        pltpu.sync_copy(nbr_hbm.at[pl.ds(base, n_per)], idx_nbr)
        pltpu.sync_copy(uiu_hbm.at[pl.ds(base, n_per)], idx_uiu)
        pltpu.sync_copy(uui_hbm.at[pl.ds(base, n_per)], idx_uui)
        pltpu.sync_copy(uid_hbm.at[pl.ds(base, n_per)], uid_v)
        pltpu.sync_copy(wpack_hbm, wv)
        pltpu.sync_copy(gx_hbm, gxm)
        pltpu.async_copy(user_hbm.at[uid_v], selfr, sem).wait()

        iota = lax.iota(i32, NLANE)
        zero16 = jnp.zeros((NLANE,), f32)
        mask8 = iota < 8

        tree_idx = []
        for r, w in ((1, 16), (2, 8), (4, 4), (8, 2)):
            h = w // 2
            kq = iota // h
            t = iota % h
            ia = jnp.minimum(kq * w + t, NLANE - 1)
            ib = jnp.clip((kq - r) * w + t, 0, NLANE - 1)
            tree_idx.append((ia, jnp.minimum(ia + h, NLANE - 1),
                             ib, jnp.minimum(ib + h, NLANE - 1)))

        def perm(x, idx):
            return jnp.take_along_axis(x, idx, axis=0,
                                       mode="promise_in_bounds")

        def lane_sums(vecs):
            for lv in range(4):
                ia, iah, ib, ibh = tree_idx[lv]
                vecs = [jnp.where(mask8,
                                  perm(vecs[2 * k], ia) + perm(vecs[2 * k], iah),
                                  perm(vecs[2 * k + 1], ib)
                                  + perm(vecs[2 * k + 1], ibh))
                        for k in range(len(vecs) // 2)]
            return vecs[0]

        def streams(i, b):
            return (
                (user_hbm.at[idx_nbr.at[i]], r_nbr.at[b]),
                (item_hbm.at[idx_uiu.at[i, 0]], r_ui_i.at[b]),
                (user_hbm.at[idx_uiu.at[i, 1]], r_ui_u.at[b]),
                (user_hbm.at[idx_uui.at[i, 0]], r_uu_u.at[b]),
                (item_hbm.at[idx_uui.at[i, 1]], r_uu_i.at[b]),
            )

        def issue(i, b):
            for src, dst in streams(i, b):
                pltpu.async_copy(src, dst, dsem.at[b])

        def drain(i, b):
            for src, dst in streams(i, b):
                pltpu.make_async_copy(src, dst, dsem.at[b]).wait()

        outs = (uu_out, uiu_out, uui_out)

        def out_streams(i):
            return [(ostage.at[p], outs[p].at[pl.ds(base + i - (OCH - 1), OCH)])
                    for p in range(3)]

        def compute(i, b):
            io = lax.rem(i, OCH)
            @pl.when(jnp.logical_and(io == 0, i > 0))
            def _():
                for src, dst in out_streams(i - 1):
                    pltpu.make_async_copy(src, dst, osem).wait()

            sv = [selfr[i, pl.ds(16 * j, 16)] for j in range(4)]
            ifull = jnp.full((NLANE,), i, i32)

            def presum(lp, carry):
                for dstb, srcb in ((r_ui_i, r_ui_u), (r_uu_u, r_uu_i)):
                    for j in range(4):
                        s = pl.ds(16 * j, 16)
                        dstb[b, lp, s] = dstb[b, lp, s] + srcb[b, lp, s]
                return carry

            lax.fori_loop(0, LNBR, presum, 0)

            for p, (bufs, midx, coef) in enumerate((
                    ((r_nbr,), None, 0.5),
                    ((r_ui_i,), idx_uiu, 1.0 / 3.0),
                    ((r_uu_u,), idx_uui, 1.0 / 3.0),
            )):
                urow = 2 * p + 1
                cself = jnp.float32(0.0)
                for j in range(4):
                    cself = cself + jnp.sum(sv[j] * wv[urow, j])

                rids = [jnp.minimum(iota + 16 * j, LNBR - 1) for j in range(4)]

                gqs = [gxm[p, 16 * q] for q in range(4)]

                def s1body(jj, carry, buf=bufs[0], gqs=gqs):
                    vecs = []
                    for cc in range(16):
                        l = jnp.minimum(16 * jj + cc, LNBR - 1)
                        s = gqs[0] * buf[b, l, pl.ds(0, 16)]
                        for q in range(1, 4):
                            s = s + gqs[q] * buf[b, l, pl.ds(16 * q, 16)]
                        vecs.append(s)
                    dstage[jj] = lane_sums(vecs)
                    return carry

                lax.fori_loop(0, 4, s1body, 0)
                d = [dstage[j] for j in range(4)]
                for j in range(4):
                    dj = d[j] + cself
                    dj = jnp.where(dj >= 0.0, dj, 0.01 * dj)
                    if midx is None:
                        miv = plsc.load_gather(idx_nbr, [ifull, rids[j]])
                    else:
                        miv = plsc.load_gather(
                            midx, [ifull, jnp.zeros((NLANE,), i32), rids[j]])
                    dj = jnp.where(miv == 0, dj - 1e8, dj)
                    dj = jnp.where(iota + 16 * j >= LNBR, -1e30, dj)
                    d[j] = dj

                mall = jnp.max(jnp.maximum(jnp.maximum(d[0], d[1]),
                                           jnp.maximum(d[2], d[3])))
                es = [jnp.exp(dj - mall) for dj in d]
                ssum = jnp.sum(es[0] + es[1] + es[2] + es[3])
                for j in range(4):
                    attnv[j] = es[j]

                def s2body(q, o, bufs=bufs):
                    av = attnv[q]
                    o = list(o)
                    for cc in range(16):
                        l = 16 * q + cc
                        a_sc = av[cc]
                        for j in range(4):
                            t = bufs[0][b, l, pl.ds(16 * j, 16)]
                            if len(bufs) == 2:
                                t = t + bufs[1][b, l, pl.ds(16 * j, 16)]
                            o[j] = o[j] + a_sc * t
                    return tuple(o)

                o = list(lax.fori_loop(0, 3, s2body,
                                       (zero16, zero16, zero16, zero16)))
                av3 = attnv[3]
                for cc in range(LNBR - 48):
                    l3 = 48 + cc
                    a_sc = av3[cc]
                    for j in range(4):
                        t = bufs[0][b, l3, pl.ds(16 * j, 16)]
                        if len(bufs) == 2:
                            t = t + bufs[1][b, l3, pl.ds(16 * j, 16)]
                        o[j] = o[j] + a_sc * t
                rinv = (jnp.full((NLANE,), coef, f32)
                        / jnp.full((NLANE,), ssum, f32))
                for j in range(4):
                    ostage[p, io, pl.ds(16 * j, 16)] = jnp.maximum(
                        coef * sv[j] + rinv * o[j], 0.0)

            @pl.when(io == OCH - 1)
            def _():
                for src, dst in out_streams(i):
                    pltpu.async_copy(src, dst, osem)

        for b in range(NBUF):
            issue(b, b)

        def body(i, carry):
            b = lax.rem(i, NBUF)
            drain(i, b)
            compute(i, b)
            inext = i + NBUF

            @pl.when(inext < n_per)
            def _():
                issue(inext, b)
            return carry

        lax.fori_loop(0, n_per, body, 0)

        for src, dst in out_streams(n_per - 1):
            pltpu.make_async_copy(src, dst, osem).wait()

    return k(user_embs, item_embs, uid, nbr, uiu, uui, wpack, gx)


def _tc_combine(uu, uiu, uui, W0, b0, W1):
    B = uu.shape[0]

    def body(uu_ref, uiu_ref, uui_ref, w0_ref, b0_ref, w1_ref, out_ref):
        w0 = w0_ref[...]
        b0 = b0_ref[...]
        w1 = w1_ref[...]
        feats = [uu_ref[...], uiu_ref[...], uui_ref[...]]
        ss = []
        for f in feats:
            h = jnp.tanh(lax.dot_general(
                f, w0, (((1,), (1,)), ((), ())),
                preferred_element_type=jnp.float32) + b0)
            hm = jnp.mean(h, axis=0, keepdims=True)
            ss.append(jnp.sum(hm * w1))
        m = jnp.maximum(jnp.maximum(ss[0], ss[1]), ss[2])
        e = [jnp.exp(x - m) for x in ss]
        tot = e[0] + e[1] + e[2]
        out = (e[0] * feats[0] + e[1] * feats[1] + e[2] * feats[2]) / tot
        out_ref[...] = jnp.maximum(out, 0.0)

    return pl.pallas_call(
        body,
        out_shape=jax.ShapeDtypeStruct((B, EDIM), jnp.float32),
    )(uu, uiu, uui, W0, b0, W1)


def kernel(user_embs, item_embs, uid, nbr, uiu, uui, W_uu, W_uiu, W_uui, W0, b0, W1):
    E = EDIM

    def gu(W, K):
        g = W[0, E:] / K
        return g, W[0, :E] + g

    g0, u0 = gu(W_uu, 2.0)
    g1, u1 = gu(W_uiu, 3.0)
    g2, u2 = gu(W_uui, 3.0)
    zeros = jnp.zeros((E,), jnp.float32)
    wpack = jnp.stack([g0, u0, g1, u1, g2, u2, zeros, zeros]).reshape(8, 4, 16)
    rot = (jnp.arange(E)[:, None] + jnp.arange(NLANE)[None, :]) % E
    gx = jnp.stack([g0[rot], g1[rot], g2[rot]])

    uu_f, uiu_f, uui_f = _sc_feats(user_embs, item_embs, uid, nbr, uiu, uui,
                                   wpack, gx)
    return _tc_combine(uu_f, uiu_f, uui_f, W0,
                       b0.reshape(1, E // 2), W1)
